# baseline (device time: 25500 ns/iter reference)
import os

import jax
import jax.numpy as jnp
from jax import lax
from jax.experimental import pallas as pl
from jax.experimental.pallas import tpu as pltpu

N_DEV = 8
N_ROUNDS = 3
N_PARTS = 3
_SKIP_COMM = os.environ.get("KERNEL_SKIP_COMM") == "1"


def kernel(x, Wg, Wu, Wd):
    m, k = x.shape
    n = Wd.shape[1]

    base = (m // N_PARTS) // 16 * 16
    sizes = [base, base, m - 2 * base]
    offs = [0, base, 2 * base]
    max_rows = max(sizes)

    def body(x_ref, wg_ref, wu_ref, wd_ref, out_ref,
             wg_v, wu_v, wd_v, w_sems,
             send_buf, recv_buf, send_sems, recv_sems):
        my = lax.axis_index("i")
        partners = [
            my ^ 1,
            (my & 4) | ((my & 3) ^ 3),
            my ^ 4,
        ]

        barrier_sem = pltpu.get_barrier_semaphore()
        for p in partners:
            pl.semaphore_signal(
                barrier_sem, inc=1,
                device_id=(p,), device_id_type=pl.DeviceIdType.MESH,
            )
        pl.semaphore_wait(barrier_sem, N_ROUNDS)

        def make_rdma(p, r):
            partner = partners[(p + r) % N_ROUNDS]
            rows = sizes[p]
            return pltpu.make_async_remote_copy(
                src_ref=send_buf.at[p, pl.ds(0, rows), :],
                dst_ref=recv_buf.at[p, r, pl.ds(0, rows), :],
                send_sem=send_sems.at[p, r],
                recv_sem=recv_sems.at[p, r],
                device_id=(partner,),
                device_id_type=pl.DeviceIdType.MESH,
            )

        copies = []
        for i, (hbm, vmem) in enumerate([(wg_ref, wg_v), (wu_ref, wu_v),
                                         (wd_ref, wd_v)]):
            c = pltpu.make_async_copy(hbm, vmem, w_sems.at[i])
            c.start()
            copies.append(c)

        xb = x_ref[:, :].astype(jnp.bfloat16)
        copies[0].wait()
        gate = jnp.dot(xb, wg_v[:, :].astype(jnp.bfloat16),
                       preferred_element_type=jnp.float32)
        copies[1].wait()
        up = jnp.dot(xb, wu_v[:, :].astype(jnp.bfloat16),
                     preferred_element_type=jnp.float32)
        h = (gate * (up * jax.nn.sigmoid(up))).astype(jnp.bfloat16)
        copies[2].wait()
        wd = wd_v[:, :].astype(jnp.bfloat16)

        parts = [None] * N_PARTS
        rdmas = {}
        for p in range(N_PARTS):
            parts[p] = jnp.dot(h[offs[p]:offs[p] + sizes[p], :], wd,
                               preferred_element_type=jnp.float32)
            if _SKIP_COMM:
                out_ref[pl.ds(offs[p], sizes[p]), :] = parts[p]
                continue
            send_buf[p, pl.ds(0, sizes[p]), :] = parts[p].astype(jnp.bfloat16)
            rdmas[p, 0] = make_rdma(p, 0)
            rdmas[p, 0].start()
        if _SKIP_COMM:
            return

        for r in range(N_ROUNDS):
            for p in range(N_PARTS):
                rdmas[p, r].wait()
                parts[p] = parts[p] + recv_buf[p, r, :sizes[p], :].astype(jnp.float32)
                if r + 1 < N_ROUNDS:
                    send_buf[p, pl.ds(0, sizes[p]), :] = parts[p].astype(jnp.bfloat16)
                    rdmas[p, r + 1] = make_rdma(p, r + 1)
                    rdmas[p, r + 1].start()
                else:
                    out_ref[pl.ds(offs[p], sizes[p]), :] = parts[p]

    return pl.pallas_call(
        body,
        out_shape=jax.ShapeDtypeStruct((m, n), jnp.float32),
        in_specs=[
            pl.BlockSpec(memory_space=pltpu.VMEM),
            pl.BlockSpec(memory_space=pl.ANY),
            pl.BlockSpec(memory_space=pl.ANY),
            pl.BlockSpec(memory_space=pl.ANY),
        ],
        out_specs=pl.BlockSpec(memory_space=pltpu.VMEM),
        scratch_shapes=[
            pltpu.VMEM(Wg.shape, jnp.float32),
            pltpu.VMEM(Wu.shape, jnp.float32),
            pltpu.VMEM(Wd.shape, jnp.float32),
            pltpu.SemaphoreType.DMA((3,)),
            pltpu.VMEM((N_PARTS, max_rows, n), jnp.bfloat16),
            pltpu.VMEM((N_PARTS, N_ROUNDS, max_rows, n), jnp.bfloat16),
            pltpu.SemaphoreType.DMA((N_PARTS, N_ROUNDS)),
            pltpu.SemaphoreType.DMA((N_PARTS, N_ROUNDS)),
        ],
        compiler_params=pltpu.CompilerParams(collective_id=0),
    )(x, Wg, Wu, Wd)


# device time: 24429 ns/iter; 1.0438x vs baseline; 1.0438x over previous
import os

import jax
import jax.numpy as jnp
from jax import lax
from jax.experimental import pallas as pl
from jax.experimental.pallas import tpu as pltpu

N_DEV = 8
N_ROUNDS = 3
N_PARTS = 3
_SKIP_COMM = os.environ.get("KERNEL_SKIP_COMM") == "1"
_ABLATE = os.environ.get("KERNEL_ABLATE", "")
_OUT_DTYPE = jnp.bfloat16 if os.environ.get("KERNEL_OUT_BF16") == "1" else jnp.float32


def kernel(x, Wg, Wu, Wd):
    m, k = x.shape
    n = Wd.shape[1]

    base = (m // N_PARTS) // 16 * 16
    sizes = [base, base, m - 2 * base]
    offs = [0, base, 2 * base]
    max_rows = max(sizes)

    def body(x_ref, wg_ref, wu_ref, wd_ref, out_ref,
             send_buf, recv_buf, send_sems, recv_sems):
        my = lax.axis_index("i")
        partners = [
            my ^ 1,
            (my & 4) | ((my & 3) ^ 3),
            my ^ 4,
        ]

        barrier_sem = pltpu.get_barrier_semaphore()
        for p in partners:
            pl.semaphore_signal(
                barrier_sem, inc=1,
                device_id=(p,), device_id_type=pl.DeviceIdType.MESH,
            )
        pl.semaphore_wait(barrier_sem, N_ROUNDS)

        def make_rdma(p, r):
            partner = partners[(p + r) % N_ROUNDS]
            rows = sizes[p]
            return pltpu.make_async_remote_copy(
                src_ref=send_buf.at[p, pl.ds(0, rows), :],
                dst_ref=recv_buf.at[p, r, pl.ds(0, rows), :],
                send_sem=send_sems.at[p, r],
                recv_sem=recv_sems.at[p, r],
                device_id=(partner,),
                device_id_type=pl.DeviceIdType.MESH,
            )

        if _ABLATE == "empty":
            out_ref[:, :] = x_ref[:, :].astype(_OUT_DTYPE)
            return
        xb = x_ref[:, :].astype(jnp.bfloat16)
        if _ABLATE == "matmul":
            h = jnp.concatenate([xb, xb], axis=1)
        else:
            gate = jnp.dot(xb, wg_ref[:, :].astype(jnp.bfloat16),
                           preferred_element_type=jnp.float32)
            up = jnp.dot(xb, wu_ref[:, :].astype(jnp.bfloat16),
                         preferred_element_type=jnp.float32)
            if _ABLATE == "silu":
                h = (gate * up).astype(jnp.bfloat16)
            else:
                h = (gate * (up * jax.nn.sigmoid(up))).astype(jnp.bfloat16)
        wd = wd_ref[:, :].astype(jnp.bfloat16)

        parts = [None] * N_PARTS
        rdmas = {}
        for p in range(N_PARTS):
            parts[p] = jnp.dot(h[offs[p]:offs[p] + sizes[p], :], wd,
                               preferred_element_type=jnp.float32)
            if _SKIP_COMM:
                out_ref[pl.ds(offs[p], sizes[p]), :] = parts[p].astype(_OUT_DTYPE)
                continue
            send_buf[p, pl.ds(0, sizes[p]), :] = parts[p].astype(jnp.bfloat16)
            rdmas[p, 0] = make_rdma(p, 0)
            rdmas[p, 0].start()
        if _SKIP_COMM:
            return

        for r in range(N_ROUNDS):
            for p in range(N_PARTS):
                rdmas[p, r].wait()
                parts[p] = parts[p] + recv_buf[p, r, :sizes[p], :].astype(jnp.float32)
                if r + 1 < N_ROUNDS:
                    send_buf[p, pl.ds(0, sizes[p]), :] = parts[p].astype(jnp.bfloat16)
                    rdmas[p, r + 1] = make_rdma(p, r + 1)
                    rdmas[p, r + 1].start()
                else:
                    out_ref[pl.ds(offs[p], sizes[p]), :] = parts[p].astype(_OUT_DTYPE)

    return pl.pallas_call(
        body,
        out_shape=jax.ShapeDtypeStruct((m, n), _OUT_DTYPE),
        in_specs=[pl.BlockSpec(memory_space=pltpu.VMEM)] * 4,
        out_specs=pl.BlockSpec(memory_space=pltpu.VMEM),
        scratch_shapes=[
            pltpu.VMEM((N_PARTS, max_rows, n), jnp.bfloat16),
            pltpu.VMEM((N_PARTS, N_ROUNDS, max_rows, n), jnp.bfloat16),
            pltpu.SemaphoreType.DMA((N_PARTS, N_ROUNDS)),
            pltpu.SemaphoreType.DMA((N_PARTS, N_ROUNDS)),
        ],
        compiler_params=pltpu.CompilerParams(collective_id=0),
    )(x, Wg, Wu, Wd)
